# Initial kernel scaffold; baseline (speedup 1.0000x reference)
import jax
import jax.numpy as jnp
from jax.experimental import pallas as pl


def kernel(uid, iid, user_genres, user_genres_offset, uid_emb_w, iid_emb_w, genres_emb_w, lr_w, lr_b):
    raise NotImplementedError("write your pallas kernel here")


# speedup vs baseline: 1.2807x; 1.2807x over previous
import functools
import jax
import jax.numpy as jnp
from jax import lax
from jax.experimental import pallas as pl
from jax.experimental.pallas import tpu as pltpu
from jax.experimental.pallas import tpu_sc as plsc

B = 16384
EMB = 32
NC, NS, L = 2, 16, 16
NW = NC * NS
BPW = B // NW
NCHUNK = 4
CHUNK = BPW // NCHUNK
GROUPS = BPW // L


def _sc_body(uid_hbm, iid_hbm, gen_hbm, wb_hbm, uemb, iemb, gemb, out_hbm,
             uidx, iidx, gidx, urows, irows, grows, wb_v, out_v, sem):
    wid = lax.axis_index("s") * NC + lax.axis_index("c")
    base4 = wid * NCHUNK

    pltpu.sync_copy(wb_hbm, wb_v)
    pltpu.sync_copy(uid_hbm.at[pl.ds(base4, NCHUNK)], uidx)
    pltpu.sync_copy(iid_hbm.at[pl.ds(base4, NCHUNK)], iidx)
    pltpu.sync_copy(gen_hbm.at[pl.ds(base4, NCHUNK)], gidx)

    copies = []
    for j in range(NCHUNK):
        dst = pl.ds(j * CHUNK, CHUNK)
        copies.append(pltpu.async_copy(uemb.at[uidx.at[j]], urows.at[dst], sem))
        copies.append(pltpu.async_copy(iemb.at[iidx.at[j]], irows.at[dst], sem))
        copies.append(pltpu.async_copy(gemb.at[gidx.at[j]], grows.at[dst], sem))
    for c in copies:
        c.wait()

    wchunks = [wb_v[pl.ds(16 * k, L)] for k in range(6)]
    zeros16 = jnp.zeros((L,), jnp.int32)
    bias = wb_v[pl.ds(96, L)].at[zeros16].get(mode="promise_in_bounds")
    lane = lax.iota(jnp.int32, L)

    def group(g, carry):
        bidx = jnp.full((L,), g * L, jnp.int32) + lane
        acc = bias
        for t, rows_ref in enumerate((urows, irows, grows)):
            for d in range(EMB):
                col = jnp.full((L,), d, jnp.int32)
                vals = plsc.load_gather(rows_ref, [bidx, col])
                wsp = wchunks[2 * t + d // L].at[
                    jnp.full((L,), d % L, jnp.int32)
                ].get(mode="promise_in_bounds")
                acc = acc + vals * wsp
        y = 1.0 / (1.0 + jnp.exp(-acc))
        out_v[pl.ds(g * L, L)] = y
        return carry

    lax.fori_loop(0, GROUPS, group, 0)
    pltpu.sync_copy(out_v, out_hbm.at[pl.ds(wid * BPW, BPW)])


@jax.jit
def _run(uid2d, iid2d, gen2d, wb, uemb, iemb, gemb):
    mesh = plsc.VectorSubcoreMesh(core_axis_name="c", subcore_axis_name="s", num_cores=2, num_subcores=16)
    f = pl.kernel(
        _sc_body,
        out_type=jax.ShapeDtypeStruct((B,), jnp.float32),
        mesh=mesh,
        compiler_params=pltpu.CompilerParams(
            needs_layout_passes=False, use_tc_tiling_on_sc=False),
        scratch_types=[
            pltpu.VMEM((NCHUNK, CHUNK), jnp.int32),
            pltpu.VMEM((NCHUNK, CHUNK), jnp.int32),
            pltpu.VMEM((NCHUNK, CHUNK), jnp.int32),
            pltpu.VMEM((BPW, EMB), jnp.float32),
            pltpu.VMEM((BPW, EMB), jnp.float32),
            pltpu.VMEM((BPW, EMB), jnp.float32),
            pltpu.VMEM((128,), jnp.float32),
            pltpu.VMEM((BPW,), jnp.float32),
            pltpu.SemaphoreType.DMA,
        ],
    )
    return f(uid2d, iid2d, gen2d, wb, uemb, iemb, gemb)


def kernel(uid, iid, user_genres, user_genres_offset, uid_emb_w, iid_emb_w,
           genres_emb_w, lr_w, lr_b):
    uid2d = uid.astype(jnp.int32).reshape(B // CHUNK, CHUNK)
    iid2d = iid.astype(jnp.int32).reshape(B // CHUNK, CHUNK)
    gen2d = user_genres.astype(jnp.int32).reshape(B // CHUNK, CHUNK)
    wb = jnp.concatenate([lr_w.reshape(-1), lr_b.reshape(-1),
                          jnp.zeros((128 - 3 * EMB - 1,), jnp.float32)])
    y = _run(uid2d, iid2d, gen2d, wb, uid_emb_w, iid_emb_w, genres_emb_w)
    return y.reshape(B, 1)


# speedup vs baseline: 1.8385x; 1.4355x over previous
import jax
import jax.numpy as jnp
from jax import lax
from jax.experimental import pallas as pl
from jax.experimental.pallas import tpu as pltpu
from jax.experimental.pallas import tpu_sc as plsc

B = 16384
EMB = 32
NC, NS, L = 2, 16, 16
NW = NC * NS
BPW = B // NW
GROUPS = BPW // L


def _sc_body(uid_hbm, iid_hbm, gen_hbm, wb_hbm, uemb, iemb, gemb, out_hbm,
             uidx, iidx, gidx, ubuf, ibuf, gbuf, wb_v, out_v,
             semu, semi, semg):
    wid = lax.axis_index("s") * NC + lax.axis_index("c")
    base = wid * BPW

    pltpu.sync_copy(wb_hbm, wb_v)
    pltpu.sync_copy(uid_hbm.at[pl.ds(base, BPW)], uidx)
    pltpu.sync_copy(iid_hbm.at[pl.ds(base, BPW)], iidx)
    pltpu.sync_copy(gen_hbm.at[pl.ds(base, BPW)], gidx)

    wchunks = [wb_v[pl.ds(16 * k, L)] for k in range(6)]
    zeros16 = jnp.zeros((L,), jnp.int32)
    bias = wb_v[pl.ds(96, L)].at[zeros16].get(mode="promise_in_bounds")
    lane = lax.iota(jnp.int32, L)

    def group(g, carry):
        k0 = g * L
        uvec = uidx[pl.ds(k0, L)]
        ivec = iidx[pl.ds(k0, L)]
        gvec = gidx[pl.ds(k0, L)]
        for k in range(L):
            pltpu.async_copy(uemb.at[pl.ds(uvec[k], 1)],
                             ubuf.at[pl.ds(k, 1)], semu)
            pltpu.async_copy(iemb.at[pl.ds(ivec[k], 1)],
                             ibuf.at[pl.ds(k, 1)], semi)
            pltpu.async_copy(gemb.at[pl.ds(gvec[k], 1)],
                             gbuf.at[pl.ds(k, 1)], semg)
        pltpu.make_async_copy(uemb.at[pl.ds(0, L)], ubuf, semu).wait()
        pltpu.make_async_copy(iemb.at[pl.ds(0, L)], ibuf, semi).wait()
        pltpu.make_async_copy(gemb.at[pl.ds(0, L)], gbuf, semg).wait()

        acc = bias
        for t, buf in enumerate((ubuf, ibuf, gbuf)):
            for d in range(EMB):
                vals = plsc.load_gather(buf, [lane, jnp.full((L,), d, jnp.int32)])
                wsp = wchunks[2 * t + d // L].at[
                    jnp.full((L,), d % L, jnp.int32)
                ].get(mode="promise_in_bounds")
                acc = acc + vals * wsp
        y = 1.0 / (1.0 + jnp.exp(-acc))
        out_v[pl.ds(k0, L)] = y
        return carry

    lax.fori_loop(0, GROUPS, group, 0)
    pltpu.sync_copy(out_v, out_hbm.at[pl.ds(base, BPW)])


@jax.jit
def _run(uid1d, iid1d, gen1d, wb, uemb, iemb, gemb):
    mesh = plsc.VectorSubcoreMesh(core_axis_name="c", subcore_axis_name="s",
                                  num_cores=NC, num_subcores=NS)
    f = pl.kernel(
        _sc_body,
        out_type=jax.ShapeDtypeStruct((B,), jnp.float32),
        mesh=mesh,
        compiler_params=pltpu.CompilerParams(needs_layout_passes=False),
        scratch_types=[
            pltpu.VMEM((BPW,), jnp.int32),
            pltpu.VMEM((BPW,), jnp.int32),
            pltpu.VMEM((BPW,), jnp.int32),
            pltpu.VMEM((L, EMB), jnp.float32),
            pltpu.VMEM((L, EMB), jnp.float32),
            pltpu.VMEM((L, EMB), jnp.float32),
            pltpu.VMEM((128,), jnp.float32),
            pltpu.VMEM((BPW,), jnp.float32),
            pltpu.SemaphoreType.DMA,
            pltpu.SemaphoreType.DMA,
            pltpu.SemaphoreType.DMA,
        ],
    )
    return f(uid1d, iid1d, gen1d, wb, uemb, iemb, gemb)


def kernel(uid, iid, user_genres, user_genres_offset, uid_emb_w, iid_emb_w,
           genres_emb_w, lr_w, lr_b):
    wb = jnp.concatenate([lr_w.reshape(-1), lr_b.reshape(-1),
                          jnp.zeros((128 - 3 * EMB - 1,), jnp.float32)])
    y = _run(uid.astype(jnp.int32), iid.astype(jnp.int32),
             user_genres.astype(jnp.int32), wb,
             uid_emb_w, iid_emb_w, genres_emb_w)
    return y.reshape(B, 1)


# speedup vs baseline: 3.9634x; 2.1558x over previous
import jax
import jax.numpy as jnp
from jax import lax
from jax.experimental import pallas as pl
from jax.experimental.pallas import tpu as pltpu
from jax.experimental.pallas import tpu_sc as plsc

B = 16384
EMB = 32
NC, NS, L = 2, 16, 16
NW = NC * NS
BPW = B // NW
GROUPS = BPW // L
NRING = 4
LAG = 2


def _sc_body(uid_hbm, iid_hbm, gen_hbm, wb_hbm, uembT, iembT, gembT, out_hbm,
             uidx, iidx, gidx, gtab, ubuf, ibuf, wb_v, out_v, *ring):
    uslabs = ring[0:NRING]
    islabs = ring[NRING:2 * NRING]
    usems = ring[2 * NRING:3 * NRING]
    isems = ring[3 * NRING:4 * NRING]

    wid = lax.axis_index("s") * NC + lax.axis_index("c")
    base = wid * BPW

    pltpu.sync_copy(wb_hbm, wb_v)
    pltpu.sync_copy(gembT, gtab)
    pltpu.sync_copy(uid_hbm.at[pl.ds(base, BPW)], uidx)
    pltpu.sync_copy(iid_hbm.at[pl.ds(base, BPW)], iidx)
    pltpu.sync_copy(gen_hbm.at[pl.ds(base, BPW)], gidx)

    wchunks = [wb_v[pl.ds(16 * k, L)] for k in range(6)]
    zeros16 = jnp.zeros((L,), jnp.int32)
    bias = wb_v[pl.ds(96, L)].at[zeros16].get(mode="promise_in_bounds")
    lane = lax.iota(jnp.int32, L)

    def fire(vec, k, slab, sem, table):
        col0 = pl.multiple_of(vec[k] & ~(L * 8 - 1), 128)
        pltpu.async_copy(table.at[:, pl.ds(col0, 128)], slab, sem)

    def drain(slab, sem, table):
        pltpu.make_async_copy(table.at[:, pl.ds(0, 128)], slab, sem).wait()

    def group(g, carry):
        k0 = g * L
        uvec = uidx[pl.ds(k0, L)]
        ivec = iidx[pl.ds(k0, L)]
        gvec = gidx[pl.ds(k0, L)]

        for k in range(LAG):
            fire(uvec, k, uslabs[k % NRING], usems[k % NRING], uembT)
            fire(ivec, k, islabs[k % NRING], isems[k % NRING], iembT)

        for k in range(L):
            if k + LAG < L:
                s = (k + LAG) % NRING
                fire(uvec, k + LAG, uslabs[s], usems[s], uembT)
                fire(ivec, k + LAG, islabs[s], isems[s], iembT)
            s = k % NRING
            drain(uslabs[s], usems[s], uembT)
            drain(islabs[s], isems[s], iembT)
            uoff = jnp.full((L,), uvec[k] & (128 - 1), jnp.int32)
            ioff = jnp.full((L,), ivec[k] & (128 - 1), jnp.int32)
            ubuf[k, pl.ds(0, L)] = plsc.load_gather(uslabs[s], [lane, uoff])
            ubuf[k, pl.ds(L, L)] = plsc.load_gather(uslabs[s], [lane + L, uoff])
            ibuf[k, pl.ds(0, L)] = plsc.load_gather(islabs[s], [lane, ioff])
            ibuf[k, pl.ds(L, L)] = plsc.load_gather(islabs[s], [lane + L, ioff])

        acc = bias
        for t, buf in enumerate((ubuf, ibuf)):
            for d in range(EMB):
                vals = plsc.load_gather(buf, [lane, jnp.full((L,), d, jnp.int32)])
                wsp = wchunks[2 * t + d // L].at[
                    jnp.full((L,), d % L, jnp.int32)
                ].get(mode="promise_in_bounds")
                acc = acc + vals * wsp
        for d in range(EMB):
            vals = plsc.load_gather(gtab, [jnp.full((L,), d, jnp.int32), gvec])
            wsp = wchunks[4 + d // L].at[
                jnp.full((L,), d % L, jnp.int32)
            ].get(mode="promise_in_bounds")
            acc = acc + vals * wsp

        y = 1.0 / (1.0 + jnp.exp(-acc))
        out_v[pl.ds(k0, L)] = y
        return carry

    lax.fori_loop(0, GROUPS, group, 0)
    pltpu.sync_copy(out_v, out_hbm.at[pl.ds(base, BPW)])


@jax.jit
def _run(uid1d, iid1d, gen1d, wb, uembT, iembT, gembT):
    mesh = plsc.VectorSubcoreMesh(core_axis_name="c", subcore_axis_name="s",
                                  num_cores=NC, num_subcores=NS)
    scratch = [
        pltpu.VMEM((BPW,), jnp.int32),
        pltpu.VMEM((BPW,), jnp.int32),
        pltpu.VMEM((BPW,), jnp.int32),
        pltpu.VMEM((EMB, 1000), jnp.float32),
        pltpu.VMEM((L, EMB), jnp.float32),
        pltpu.VMEM((L, EMB), jnp.float32),
        pltpu.VMEM((128,), jnp.float32),
        pltpu.VMEM((BPW,), jnp.float32),
    ]
    scratch += [pltpu.VMEM((EMB, 128), jnp.float32) for _ in range(2 * NRING)]
    scratch += [pltpu.SemaphoreType.DMA for _ in range(2 * NRING)]
    f = pl.kernel(
        _sc_body,
        out_type=jax.ShapeDtypeStruct((B,), jnp.float32),
        mesh=mesh,
        compiler_params=pltpu.CompilerParams(needs_layout_passes=False),
        scratch_types=scratch,
    )
    return f(uid1d, iid1d, gen1d, wb, uembT, iembT, gembT)


def kernel(uid, iid, user_genres, user_genres_offset, uid_emb_w, iid_emb_w,
           genres_emb_w, lr_w, lr_b):
    wb = jnp.concatenate([lr_w.reshape(-1), lr_b.reshape(-1),
                          jnp.zeros((128 - 3 * EMB - 1,), jnp.float32)])
    y = _run(uid.astype(jnp.int32), iid.astype(jnp.int32),
             user_genres.astype(jnp.int32), wb,
             uid_emb_w.T, iid_emb_w.T, genres_emb_w.T)
    return y.reshape(B, 1)


# speedup vs baseline: 4.3886x; 1.1073x over previous
import jax
import jax.numpy as jnp
from jax import lax
from jax.experimental import pallas as pl
from jax.experimental.pallas import tpu as pltpu
from jax.experimental.pallas import tpu_sc as plsc

B = 16384
EMB = 32
NC, NS, L = 2, 16, 16
NW = NC * NS
BPW = B // NW
GROUPS = BPW // L
NRING = 4
LAG = 3


def _sc_body(uid_hbm, iid_hbm, gen_hbm, wb_hbm, uembT, iembT, gembT, out_hbm,
             uidx, iidx, gidx, gtab, ubuf, ibuf, wb_v, out_v, *ring):
    uslabs = ring[0:NRING]
    islabs = ring[NRING:2 * NRING]
    usems = ring[2 * NRING:3 * NRING]
    isems = ring[3 * NRING:4 * NRING]

    wid = lax.axis_index("s") * NC + lax.axis_index("c")
    base = wid * BPW

    pltpu.sync_copy(wb_hbm, wb_v)
    pltpu.sync_copy(gembT, gtab)
    pltpu.sync_copy(uid_hbm.at[pl.ds(base, BPW)], uidx)
    pltpu.sync_copy(iid_hbm.at[pl.ds(base, BPW)], iidx)
    pltpu.sync_copy(gen_hbm.at[pl.ds(base, BPW)], gidx)

    wchunks = [wb_v[pl.ds(16 * k, L)] for k in range(6)]
    zeros16 = jnp.zeros((L,), jnp.int32)
    bias = wb_v[pl.ds(96, L)].at[zeros16].get(mode="promise_in_bounds")
    lane = lax.iota(jnp.int32, L)

    def fire(vec, k, slab, sem, table):
        col0 = pl.multiple_of(vec[k] & ~(L * 8 - 1), 128)
        pltpu.async_copy(table.at[:, pl.ds(col0, 128)], slab, sem)

    def drain(slab, sem, table):
        pltpu.make_async_copy(table.at[:, pl.ds(0, 128)], slab, sem).wait()

    def group(g, carry):
        k0 = g * L
        uvec = uidx[pl.ds(k0, L)]
        ivec = iidx[pl.ds(k0, L)]
        gvec = gidx[pl.ds(k0, L)]

        for k in range(LAG):
            fire(uvec, k, uslabs[k % NRING], usems[k % NRING], uembT)
            fire(ivec, k, islabs[k % NRING], isems[k % NRING], iembT)

        for k in range(L):
            if k + LAG < L:
                s = (k + LAG) % NRING
                fire(uvec, k + LAG, uslabs[s], usems[s], uembT)
                fire(ivec, k + LAG, islabs[s], isems[s], iembT)
            s = k % NRING
            drain(uslabs[s], usems[s], uembT)
            drain(islabs[s], isems[s], iembT)
            uoff = jnp.full((L,), uvec[k] & (128 - 1), jnp.int32)
            ioff = jnp.full((L,), ivec[k] & (128 - 1), jnp.int32)
            ubuf[k, pl.ds(0, L)] = plsc.load_gather(uslabs[s], [lane, uoff])
            ubuf[k, pl.ds(L, L)] = plsc.load_gather(uslabs[s], [lane + L, uoff])
            ibuf[k, pl.ds(0, L)] = plsc.load_gather(islabs[s], [lane, ioff])
            ibuf[k, pl.ds(L, L)] = plsc.load_gather(islabs[s], [lane + L, ioff])

        acc = bias
        for t, buf in enumerate((ubuf, ibuf)):
            for d in range(EMB):
                vals = plsc.load_gather(buf, [lane, jnp.full((L,), d, jnp.int32)])
                wsp = wchunks[2 * t + d // L].at[
                    jnp.full((L,), d % L, jnp.int32)
                ].get(mode="promise_in_bounds")
                acc = acc + vals * wsp
        for d in range(EMB):
            vals = plsc.load_gather(gtab, [jnp.full((L,), d, jnp.int32), gvec])
            wsp = wchunks[4 + d // L].at[
                jnp.full((L,), d % L, jnp.int32)
            ].get(mode="promise_in_bounds")
            acc = acc + vals * wsp

        y = 1.0 / (1.0 + jnp.exp(-acc))
        out_v[pl.ds(k0, L)] = y
        return carry

    lax.fori_loop(0, GROUPS, group, 0)
    pltpu.sync_copy(out_v, out_hbm.at[pl.ds(base, BPW)])


@jax.jit
def _run(uid1d, iid1d, gen1d, wb, uembT, iembT, gembT):
    mesh = plsc.VectorSubcoreMesh(core_axis_name="c", subcore_axis_name="s",
                                  num_cores=NC, num_subcores=NS)
    scratch = [
        pltpu.VMEM((BPW,), jnp.int32),
        pltpu.VMEM((BPW,), jnp.int32),
        pltpu.VMEM((BPW,), jnp.int32),
        pltpu.VMEM((EMB, 1000), jnp.float32),
        pltpu.VMEM((L, EMB), jnp.float32),
        pltpu.VMEM((L, EMB), jnp.float32),
        pltpu.VMEM((128,), jnp.float32),
        pltpu.VMEM((BPW,), jnp.float32),
    ]
    scratch += [pltpu.VMEM((EMB, 128), jnp.float32) for _ in range(2 * NRING)]
    scratch += [pltpu.SemaphoreType.DMA for _ in range(2 * NRING)]
    f = pl.kernel(
        _sc_body,
        out_type=jax.ShapeDtypeStruct((B,), jnp.float32),
        mesh=mesh,
        compiler_params=pltpu.CompilerParams(needs_layout_passes=False),
        scratch_types=scratch,
    )
    return f(uid1d, iid1d, gen1d, wb, uembT, iembT, gembT)


def kernel(uid, iid, user_genres, user_genres_offset, uid_emb_w, iid_emb_w,
           genres_emb_w, lr_w, lr_b):
    wb = jnp.concatenate([lr_w.reshape(-1), lr_b.reshape(-1),
                          jnp.zeros((128 - 3 * EMB - 1,), jnp.float32)])
    y = _run(uid.astype(jnp.int32), iid.astype(jnp.int32),
             user_genres.astype(jnp.int32), wb,
             uid_emb_w.T, iid_emb_w.T, genres_emb_w.T)
    return y.reshape(B, 1)
